# baseline (device time: 104185 ns/iter reference)
import jax
import jax.numpy as jnp
from jax import lax
from jax.experimental import pallas as pl
from jax.experimental.pallas import tpu as pltpu

N_DEV = 4
B, SQ, D = 4, 256, 1024
HQ, HKV, DH = 8, 2, 128
G = HQ // HKV
SKV = 1024
SCALE = 0.08838834764831843
C_OROWS = HQ * SQ
C_ROWS = C_OROWS + SQ
ROWS = N_DEV * C_ROWS
BF16 = jnp.bfloat16


def kernel(x, Wq, Wo, K_ext, V_ext):
    def body(x_ref, wq_ref, wo_ref, k_ref, v_ref, out_ref,
             part, stage, out_q16, bcstage, x16s, wq16s, k16s, v16s,
             rs_send, rs_recv, bc_send, bc_recv):
        my = lax.axis_index("i")

        barrier_sem = pltpu.get_barrier_semaphore()
        for nbr in ((my - 1) % N_DEV, (my + 1) % N_DEV):
            pl.semaphore_signal(barrier_sem, inc=1, device_id=(nbr,),
                                device_id_type=pl.DeviceIdType.MESH)
        pl.semaphore_wait(barrier_sem, 2)

        x16s[:] = x_ref[:].astype(BF16)
        wq16s[:] = wq_ref[:].astype(BF16)
        k16s[:] = k_ref[:].astype(BF16)
        v16s[:] = v_ref[:].astype(BF16)
        ones_col = jnp.ones((SKV, 1), BF16)

        def compute_batch(b):
            qb16 = jnp.dot(x16s[b], wq16s[:],
                           preferred_element_type=jnp.float32).astype(BF16)
            r0 = b * C_ROWS
            for h in range(HQ):
                g = h // G
                s = lax.dot_general(
                    qb16[:, h * DH:(h + 1) * DH], k16s[b, :, g, :],
                    (((1,), (1,)), ((), ())),
                    preferred_element_type=jnp.float32)
                p16 = jnp.exp(s * SCALE).astype(BF16)
                o = jnp.dot(p16, v16s[b, :, g, :],
                            preferred_element_type=jnp.float32)
                l = jnp.dot(p16, ones_col,
                            preferred_element_type=jnp.float32)
                part[r0 + h * SQ:r0 + (h + 1) * SQ, :] = o.astype(BF16)
                part[r0 + C_OROWS:r0 + C_ROWS, h:h + 1] = l.astype(BF16)

        for b in range(B):
            compute_batch(b)
            for d in range(1, N_DEV):
                @pl.when(((my + d) % N_DEV) == b)
                def _(b=b, d=d):
                    pltpu.make_async_remote_copy(
                        src_ref=part.at[pl.ds(b * C_ROWS, C_ROWS), :],
                        dst_ref=stage.at[d - 1],
                        send_sem=rs_send.at[d - 1],
                        recv_sem=rs_recv.at[d - 1],
                        device_id=(b,),
                        device_id_type=pl.DeviceIdType.MESH,
                    ).start()

        for j in range(N_DEV - 1):
            pltpu.make_async_remote_copy(
                src_ref=part.at[pl.ds(0, C_ROWS), :],
                dst_ref=stage.at[j],
                send_sem=rs_send.at[j],
                recv_sem=rs_recv.at[j],
                device_id=(my,),
                device_id_type=pl.DeviceIdType.MESH,
            ).wait_recv()

        tot = (part[pl.ds(my * C_ROWS, C_ROWS), :].astype(jnp.float32)
               + stage[0].astype(jnp.float32)
               + stage[1].astype(jnp.float32)
               + stage[2].astype(jnp.float32))
        cols = []
        for h in range(HQ):
            o_blk = tot[h * SQ:(h + 1) * SQ, :]
            l_blk = tot[C_OROWS:C_ROWS, h:h + 1]
            cols.append((o_blk / l_blk).astype(BF16))
        att16 = jnp.concatenate(cols, axis=1)
        wo16 = wo_ref[:].astype(BF16)
        oq = jnp.dot(att16, wo16, preferred_element_type=jnp.float32)
        out_ref[pl.ds(my, 1)] = oq[jnp.newaxis]
        out_q16[:] = oq.astype(BF16)

        bc = []
        for d in range(1, N_DEV):
            rdma = pltpu.make_async_remote_copy(
                src_ref=out_q16,
                dst_ref=bcstage.at[d - 1],
                send_sem=bc_send.at[d - 1],
                recv_sem=bc_recv.at[d - 1],
                device_id=((my + d) % N_DEV,),
                device_id_type=pl.DeviceIdType.MESH,
            )
            rdma.start()
            bc.append(rdma)
        for j in range(N_DEV - 1):
            pltpu.make_async_remote_copy(
                src_ref=out_q16,
                dst_ref=bcstage.at[j],
                send_sem=bc_send.at[j],
                recv_sem=bc_recv.at[j],
                device_id=((my + j + 1) % N_DEV,),
                device_id_type=pl.DeviceIdType.MESH,
            ).wait_recv()
            out_ref[pl.ds((my - j - 1) % N_DEV, 1)] = \
                bcstage[j].astype(jnp.float32)[jnp.newaxis]

        for j in range(N_DEV - 1):
            pltpu.make_async_remote_copy(
                src_ref=part.at[pl.ds(0, C_ROWS), :],
                dst_ref=stage.at[j],
                send_sem=rs_send.at[j],
                recv_sem=rs_recv.at[j],
                device_id=(my,),
                device_id_type=pl.DeviceIdType.MESH,
            ).wait_send()
        for rdma in bc:
            rdma.wait_send()

    return pl.pallas_call(
        body,
        out_shape=jax.ShapeDtypeStruct((B, SQ, D), jnp.float32),
        in_specs=[pl.BlockSpec(memory_space=pltpu.VMEM)] * 5,
        out_specs=pl.BlockSpec(memory_space=pltpu.VMEM),
        scratch_shapes=[
            pltpu.VMEM((ROWS, DH), BF16),
            pltpu.VMEM((N_DEV - 1, C_ROWS, DH), BF16),
            pltpu.VMEM((SQ, D), BF16),
            pltpu.VMEM((N_DEV - 1, SQ, D), BF16),
            pltpu.VMEM((B, SQ, D), BF16),
            pltpu.VMEM((D, D), BF16),
            pltpu.VMEM((B, SKV, HKV, DH), BF16),
            pltpu.VMEM((B, SKV, HKV, DH), BF16),
            pltpu.SemaphoreType.DMA((N_DEV - 1,)),
            pltpu.SemaphoreType.DMA((N_DEV - 1,)),
            pltpu.SemaphoreType.DMA((N_DEV - 1,)),
            pltpu.SemaphoreType.DMA((N_DEV - 1,)),
        ],
        compiler_params=pltpu.CompilerParams(
            collective_id=0, vmem_limit_bytes=100 * 1024 * 1024),
    )(x, Wq, Wo, K_ext, V_ext)


# device time: 99727 ns/iter; 1.0447x vs baseline; 1.0447x over previous
import jax
import jax.numpy as jnp
from jax import lax
from jax.experimental import pallas as pl
from jax.experimental.pallas import tpu as pltpu

N_DEV = 4
B, SQ, D = 4, 256, 1024
HQ, HKV, DH = 8, 2, 128
G = HQ // HKV
SKV = 1024
SCALE = 0.08838834764831843
C_OROWS = HQ * SQ
C_ROWS = C_OROWS + SQ
ROWS = N_DEV * C_ROWS
BF16 = jnp.bfloat16


def kernel(x, Wq, Wo, K_ext, V_ext):
    def body(x_ref, wq_ref, wo_ref, k_ref, v_ref, out_ref,
             part, stage, out_q16, bcstage, x16s, wq16s, k16s, v16s,
             rs_send, rs_recv, bc_send, bc_recv):
        my = lax.axis_index("i")

        barrier_sem = pltpu.get_barrier_semaphore()
        for nbr in ((my - 1) % N_DEV, (my + 1) % N_DEV):
            pl.semaphore_signal(barrier_sem, inc=1, device_id=(nbr,),
                                device_id_type=pl.DeviceIdType.MESH)
        pl.semaphore_wait(barrier_sem, 2)

        x16s[:] = x_ref[:].astype(BF16)
        wq16s[:] = wq_ref[:].astype(BF16)
        k16s[:] = k_ref[:].astype(BF16)
        v16s[:] = v_ref[:].astype(BF16)
        ones_col = jnp.ones((SKV, 1), BF16)

        def compute_batch(b):
            qb16 = jnp.dot(x16s[b], wq16s[:],
                           preferred_element_type=jnp.float32).astype(BF16)
            r0 = b * C_ROWS
            for h in range(HQ):
                g = h // G
                s = lax.dot_general(
                    qb16[:, h * DH:(h + 1) * DH], k16s[b, :, g, :],
                    (((1,), (1,)), ((), ())),
                    preferred_element_type=jnp.float32)
                p = jnp.exp(s * SCALE)
                l = jnp.sum(p, axis=1, keepdims=True)
                o = jnp.dot(p.astype(BF16), v16s[b, :, g, :],
                            preferred_element_type=jnp.float32)
                part[r0 + h * SQ:r0 + (h + 1) * SQ, :] = o.astype(BF16)
                part[r0 + C_OROWS:r0 + C_ROWS, h:h + 1] = l.astype(BF16)

        for b in range(B):
            compute_batch(b)
            for d in range(1, N_DEV):
                @pl.when(((my + d) % N_DEV) == b)
                def _(b=b, d=d):
                    pltpu.make_async_remote_copy(
                        src_ref=part.at[pl.ds(b * C_ROWS, C_ROWS), :],
                        dst_ref=stage.at[d - 1],
                        send_sem=rs_send.at[d - 1],
                        recv_sem=rs_recv.at[d - 1],
                        device_id=(b,),
                        device_id_type=pl.DeviceIdType.MESH,
                    ).start()

        for j in range(N_DEV - 1):
            pltpu.make_async_remote_copy(
                src_ref=part.at[pl.ds(0, C_ROWS), :],
                dst_ref=stage.at[j],
                send_sem=rs_send.at[j],
                recv_sem=rs_recv.at[j],
                device_id=(my,),
                device_id_type=pl.DeviceIdType.MESH,
            ).wait_recv()

        tot = (part[pl.ds(my * C_ROWS, C_ROWS), :].astype(jnp.float32)
               + stage[0].astype(jnp.float32)
               + stage[1].astype(jnp.float32)
               + stage[2].astype(jnp.float32))
        cols = []
        for h in range(HQ):
            o_blk = tot[h * SQ:(h + 1) * SQ, :]
            l_blk = tot[C_OROWS:C_ROWS, h:h + 1]
            cols.append((o_blk / l_blk).astype(BF16))
        att16 = jnp.concatenate(cols, axis=1)
        wo16 = wo_ref[:].astype(BF16)
        oq = jnp.dot(att16, wo16, preferred_element_type=jnp.float32)
        out_ref[pl.ds(my, 1)] = oq[jnp.newaxis]
        out_q16[:] = oq.astype(BF16)

        bc = []
        for d in range(1, N_DEV):
            rdma = pltpu.make_async_remote_copy(
                src_ref=out_q16,
                dst_ref=bcstage.at[d - 1],
                send_sem=bc_send.at[d - 1],
                recv_sem=bc_recv.at[d - 1],
                device_id=((my + d) % N_DEV,),
                device_id_type=pl.DeviceIdType.MESH,
            )
            rdma.start()
            bc.append(rdma)
        for j in range(N_DEV - 1):
            pltpu.make_async_remote_copy(
                src_ref=out_q16,
                dst_ref=bcstage.at[j],
                send_sem=bc_send.at[j],
                recv_sem=bc_recv.at[j],
                device_id=((my + j + 1) % N_DEV,),
                device_id_type=pl.DeviceIdType.MESH,
            ).wait_recv()
            out_ref[pl.ds((my - j - 1) % N_DEV, 1)] = \
                bcstage[j].astype(jnp.float32)[jnp.newaxis]

        for j in range(N_DEV - 1):
            pltpu.make_async_remote_copy(
                src_ref=part.at[pl.ds(0, C_ROWS), :],
                dst_ref=stage.at[j],
                send_sem=rs_send.at[j],
                recv_sem=rs_recv.at[j],
                device_id=(my,),
                device_id_type=pl.DeviceIdType.MESH,
            ).wait_send()
        for rdma in bc:
            rdma.wait_send()

    return pl.pallas_call(
        body,
        out_shape=jax.ShapeDtypeStruct((B, SQ, D), jnp.float32),
        in_specs=[pl.BlockSpec(memory_space=pltpu.VMEM)] * 5,
        out_specs=pl.BlockSpec(memory_space=pltpu.VMEM),
        scratch_shapes=[
            pltpu.VMEM((ROWS, DH), BF16),
            pltpu.VMEM((N_DEV - 1, C_ROWS, DH), BF16),
            pltpu.VMEM((SQ, D), BF16),
            pltpu.VMEM((N_DEV - 1, SQ, D), BF16),
            pltpu.VMEM((B, SQ, D), BF16),
            pltpu.VMEM((D, D), BF16),
            pltpu.VMEM((B, SKV, HKV, DH), BF16),
            pltpu.VMEM((B, SKV, HKV, DH), BF16),
            pltpu.SemaphoreType.DMA((N_DEV - 1,)),
            pltpu.SemaphoreType.DMA((N_DEV - 1,)),
            pltpu.SemaphoreType.DMA((N_DEV - 1,)),
            pltpu.SemaphoreType.DMA((N_DEV - 1,)),
        ],
        compiler_params=pltpu.CompilerParams(
            collective_id=0, vmem_limit_bytes=100 * 1024 * 1024),
    )(x, Wq, Wo, K_ext, V_ext)


# device time: 63449 ns/iter; 1.6420x vs baseline; 1.5718x over previous
import jax
import jax.numpy as jnp
from jax import lax
from jax.experimental import pallas as pl
from jax.experimental.pallas import tpu as pltpu

N_DEV = 4
B, SQ, D = 4, 256, 1024
HQ, HKV, DH = 8, 2, 128
G = HQ // HKV
SCALE = 0.08838834764831843
C_OROWS = HQ * SQ
C_ROWS = C_OROWS + SQ
ROWS = N_DEV * C_ROWS
BF16 = jnp.bfloat16


def kernel(x, Wq, Wo, K_ext, V_ext):
    def body(x_ref, wq_ref, wo_ref, k_ref, v_ref, out_ref,
             part, stage, out_q16, bcstage,
             rs_send, rs_recv, bc_send, bc_recv):
        my = lax.axis_index("i")

        barrier_sem = pltpu.get_barrier_semaphore()
        for nbr in ((my - 1) % N_DEV, (my + 1) % N_DEV):
            pl.semaphore_signal(barrier_sem, inc=1, device_id=(nbr,),
                                device_id_type=pl.DeviceIdType.MESH)
        pl.semaphore_wait(barrier_sem, 2)

        x16 = x_ref[:].astype(BF16)
        wq16 = wq_ref[:].astype(BF16)
        k16 = k_ref[:].astype(BF16)
        v16 = v_ref[:].astype(BF16)

        for b in range(B):
            qb16 = jnp.dot(x16[b], wq16,
                           preferred_element_type=jnp.float32).astype(BF16)
            r0 = b * C_ROWS
            for h in range(HQ):
                g = h // G
                s = lax.dot_general(
                    qb16[:, h * DH:(h + 1) * DH], k16[b, :, g, :],
                    (((1,), (1,)), ((), ())),
                    preferred_element_type=jnp.float32) * SCALE
                p = jnp.exp(s)
                l = jnp.sum(p, axis=1, keepdims=True)
                o = jnp.dot(p.astype(BF16), v16[b, :, g, :],
                            preferred_element_type=jnp.float32)
                part[r0 + h * SQ:r0 + (h + 1) * SQ, :] = o.astype(BF16)
                part[r0 + C_OROWS:r0 + C_ROWS, h:h + 1] = l.astype(BF16)
            for d in range(1, N_DEV):
                @pl.when(((my + d) % N_DEV) == b)
                def _(b=b, d=d):
                    pltpu.make_async_remote_copy(
                        src_ref=part.at[pl.ds(b * C_ROWS, C_ROWS), :],
                        dst_ref=stage.at[d - 1],
                        send_sem=rs_send.at[d - 1],
                        recv_sem=rs_recv.at[d - 1],
                        device_id=(b,),
                        device_id_type=pl.DeviceIdType.MESH,
                    ).start()

        for j in range(N_DEV - 1):
            pltpu.make_async_remote_copy(
                src_ref=part.at[pl.ds(0, C_ROWS), :],
                dst_ref=stage.at[j],
                send_sem=rs_send.at[j],
                recv_sem=rs_recv.at[j],
                device_id=(my,),
                device_id_type=pl.DeviceIdType.MESH,
            ).wait_recv()

        tot = (part[pl.ds(my * C_ROWS, C_ROWS), :].astype(jnp.float32)
               + stage[0].astype(jnp.float32)
               + stage[1].astype(jnp.float32)
               + stage[2].astype(jnp.float32))
        cols = []
        for h in range(HQ):
            o_blk = tot[h * SQ:(h + 1) * SQ, :]
            l_blk = tot[C_OROWS:C_ROWS, h:h + 1]
            cols.append((o_blk / l_blk).astype(BF16))
        att16 = jnp.concatenate(cols, axis=1)
        wo16 = wo_ref[:].astype(BF16)
        oq = jnp.dot(att16, wo16, preferred_element_type=jnp.float32)
        out_ref[pl.ds(my, 1)] = oq[jnp.newaxis]
        out_q16[:] = oq.astype(BF16)

        bc = []
        for d in range(1, N_DEV):
            rdma = pltpu.make_async_remote_copy(
                src_ref=out_q16,
                dst_ref=bcstage.at[d - 1],
                send_sem=bc_send.at[d - 1],
                recv_sem=bc_recv.at[d - 1],
                device_id=((my + d) % N_DEV,),
                device_id_type=pl.DeviceIdType.MESH,
            )
            rdma.start()
            bc.append(rdma)
        for j in range(N_DEV - 1):
            pltpu.make_async_remote_copy(
                src_ref=out_q16,
                dst_ref=bcstage.at[j],
                send_sem=bc_send.at[j],
                recv_sem=bc_recv.at[j],
                device_id=((my + j + 1) % N_DEV,),
                device_id_type=pl.DeviceIdType.MESH,
            ).wait_recv()
            out_ref[pl.ds((my - j - 1) % N_DEV, 1)] = \
                bcstage[j].astype(jnp.float32)[jnp.newaxis]

        for j in range(N_DEV - 1):
            pltpu.make_async_remote_copy(
                src_ref=part.at[pl.ds(0, C_ROWS), :],
                dst_ref=stage.at[j],
                send_sem=rs_send.at[j],
                recv_sem=rs_recv.at[j],
                device_id=(my,),
                device_id_type=pl.DeviceIdType.MESH,
            ).wait_send()
        for rdma in bc:
            rdma.wait_send()

    return pl.pallas_call(
        body,
        out_shape=jax.ShapeDtypeStruct((B, SQ, D), jnp.float32),
        in_specs=[pl.BlockSpec(memory_space=pltpu.VMEM)] * 5,
        out_specs=pl.BlockSpec(memory_space=pltpu.VMEM),
        scratch_shapes=[
            pltpu.VMEM((ROWS, DH), BF16),
            pltpu.VMEM((N_DEV - 1, C_ROWS, DH), BF16),
            pltpu.VMEM((SQ, D), BF16),
            pltpu.VMEM((N_DEV - 1, SQ, D), BF16),
            pltpu.SemaphoreType.DMA((N_DEV - 1,)),
            pltpu.SemaphoreType.DMA((N_DEV - 1,)),
            pltpu.SemaphoreType.DMA((N_DEV - 1,)),
            pltpu.SemaphoreType.DMA((N_DEV - 1,)),
        ],
        compiler_params=pltpu.CompilerParams(
            collective_id=0, vmem_limit_bytes=100 * 1024 * 1024),
    )(x, Wq, Wo, K_ext, V_ext)
